# baseline (speedup 1.0000x reference)
import jax
import jax.numpy as jnp
from jax.experimental import pallas as pl
from jax.experimental.pallas import tpu as pltpu


def _copy_body(u_in, i_in, u_out, i_out, u_sem, i_sem):
    cu = pltpu.make_async_copy(u_in, u_out, u_sem)
    ci = pltpu.make_async_copy(i_in, i_out, i_sem)
    cu.start()
    ci.start()
    cu.wait()
    ci.wait()


def kernel(user_emb, item_emb):
    return pl.pallas_call(
        _copy_body,
        in_specs=[
            pl.BlockSpec(memory_space=pltpu.MemorySpace.HBM),
            pl.BlockSpec(memory_space=pltpu.MemorySpace.HBM),
        ],
        out_specs=[
            pl.BlockSpec(memory_space=pltpu.MemorySpace.HBM),
            pl.BlockSpec(memory_space=pltpu.MemorySpace.HBM),
        ],
        out_shape=[
            jax.ShapeDtypeStruct(user_emb.shape, user_emb.dtype),
            jax.ShapeDtypeStruct(item_emb.shape, item_emb.dtype),
        ],
        scratch_shapes=[pltpu.SemaphoreType.DMA, pltpu.SemaphoreType.DMA],
    )(user_emb, item_emb)


# speedup vs baseline: 19.0756x; 19.0756x over previous
import jax
import jax.numpy as jnp
from jax.experimental import pallas as pl
from jax.experimental.pallas import tpu as pltpu

GRID = 100
U_BLK = 1_000_000 // GRID
I_BLK = 100_000 // GRID


def _copy_body(u_in, i_in, u_out, i_out):
    u_out[...] = u_in[...]
    i_out[...] = i_in[...]


def kernel(user_emb, item_emb):
    return pl.pallas_call(
        _copy_body,
        grid=(GRID,),
        in_specs=[
            pl.BlockSpec((U_BLK, 16), lambda i: (i, 0)),
            pl.BlockSpec((I_BLK, 16), lambda i: (i, 0)),
        ],
        out_specs=[
            pl.BlockSpec((U_BLK, 16), lambda i: (i, 0)),
            pl.BlockSpec((I_BLK, 16), lambda i: (i, 0)),
        ],
        out_shape=[
            jax.ShapeDtypeStruct(user_emb.shape, user_emb.dtype),
            jax.ShapeDtypeStruct(item_emb.shape, item_emb.dtype),
        ],
    )(user_emb, item_emb)
